# Initial kernel scaffold; baseline (speedup 1.0000x reference)
import jax
import jax.numpy as jnp
from jax.experimental import pallas as pl


def kernel(inp, W):
    raise NotImplementedError("write your pallas kernel here")


# speedup vs baseline: 2.7235x; 2.7235x over previous
import functools

import jax
import jax.numpy as jnp
from jax import lax
from jax.experimental import pallas as pl
from jax.experimental.pallas import tpu as pltpu

MAX_SPIKE_TIME = 100.0
TH = 1.0
BIT = 8
K = 3
PAD = 1
IN_CH = 16
OUT_CH = 32
F = K * K * IN_CH


def _quantize(w):
    w = jnp.tanh(w)
    alpha = jnp.max(jnp.abs(w))
    q = 2.0 ** (BIT - 1) - 1.0
    w = jnp.clip(w / alpha, -1.0, 1.0) * q
    return jnp.round(w) * alpha / q


def _tconv_block(v_ref, w_ref, o_ref):
    np_ = v_ref.shape[0]
    v = v_ref[...]
    v = jnp.where(v < 0.1, MAX_SPIKE_TIME, v)

    wqt = w_ref[...]

    vk = v[:, :, None]
    vj = v[:, None, :]
    jj = lax.broadcasted_iota(jnp.int32, (np_, F, F), 2)
    kk = lax.broadcasted_iota(jnp.int32, (np_, F, F), 1)
    p = (vj < vk) | ((vj == vk) & (jj <= kk))
    pf = p.astype(jnp.float32)

    ws = jax.lax.dot_general(
        pf.reshape(np_ * F, F), wqt, (((1,), (0,)), ((), ())),
        precision=lax.Precision.HIGHEST,
        preferred_element_type=jnp.float32).reshape(np_, F, OUT_CH)
    iws = jax.lax.dot_general(
        (pf * vj).reshape(np_ * F, F), wqt, (((1,), (0,)), ((), ())),
        precision=lax.Precision.HIGHEST,
        preferred_element_type=jnp.float32).reshape(np_, F, OUT_CH)

    nxt = jnp.min(jnp.where(p, 1e30, jnp.broadcast_to(vj, (np_, F, F))),
                  axis=-1)
    rank = jnp.sum(pf, axis=-1)
    nxt = jnp.where(rank >= float(F), 1.0, nxt)

    d = jnp.clip(ws - TH, 1e-10, 1e10)
    oa = iws / d
    cand = jnp.where(ws < TH, MAX_SPIKE_TIME, oa)
    cand = jnp.where(cand < vk, MAX_SPIKE_TIME, cand)
    cand = jnp.where(cand > nxt[:, :, None], MAX_SPIKE_TIME, cand)
    o_ref[...] = jnp.min(cand, axis=-2)


def _unfold_patches(inp):
    b, c, h, w = inp.shape
    xp = jnp.pad(inp, ((0, 0), (0, 0), (PAD, PAD), (PAD, PAD)))
    pats = []
    for i in range(K):
        for j in range(K):
            pats.append(xp[:, :, i:i + h, j:j + w])
    p = jnp.stack(pats, axis=2)
    p = p.reshape(b, c * K * K, h * w)
    return jnp.transpose(p, (0, 2, 1)).reshape(b * h * w, c * K * K)


@jax.jit
def kernel(inp, W):
    b, c, h, w = inp.shape
    l = h * w
    v = _unfold_patches(inp)
    wqt = _quantize(W).T
    npos = b * l
    blk = 32
    grid = npos // blk
    out = pl.pallas_call(
        _tconv_block,
        grid=(grid,),
        in_specs=[
            pl.BlockSpec((blk, F), lambda i: (i, 0)),
            pl.BlockSpec((F, OUT_CH), lambda i: (0, 0)),
        ],
        out_specs=pl.BlockSpec((blk, OUT_CH), lambda i: (i, 0)),
        out_shape=jax.ShapeDtypeStruct((npos, OUT_CH), jnp.float32),
    )(v, wqt)
    out = out.reshape(b, l, OUT_CH)
    return jnp.transpose(out, (0, 2, 1)).reshape(b, OUT_CH, h, w)


# speedup vs baseline: 6.2151x; 2.2820x over previous
import functools

import jax
import jax.numpy as jnp
from jax import lax
from jax.experimental import pallas as pl
from jax.experimental.pallas import tpu as pltpu
from jax.experimental.pallas import tpu_sc as plsc

MAX_SPIKE_TIME = 100.0
TH = 1.0
BIT = 8
K = 3
PAD = 1
IN_CH = 16
OUT_CH = 32
F = K * K * IN_CH
LANES = 16
NW = 32
NPOS = 2 * 32 * 32
GROUPS = NPOS // LANES
GPW = GROUPS // NW
QCH = 8


def _quantize(w):
    w = jnp.tanh(w)
    alpha = jnp.max(jnp.abs(w))
    q = 2.0 ** (BIT - 1) - 1.0
    w = jnp.clip(w / alpha, -1.0, 1.0) * q
    return jnp.round(w) * alpha / q


def _sc_body(vt_hbm, wq_hbm, out_hbm, vt_v, cnt_v, ord_v, vs_v, wq_v,
             out_v):
    cid = lax.axis_index("c")
    sid = lax.axis_index("s")
    wid = sid * 2 + cid
    pltpu.sync_copy(wq_hbm, wq_v)
    lane = lax.iota(jnp.int32, LANES)
    zero16 = jnp.zeros((LANES,), jnp.int32)
    zero16f = jnp.zeros((LANES,), jnp.float32)

    def group_body(j, _):
        g = wid * GPW + j
        pltpu.sync_copy(vt_hbm.at[g], vt_v)

        def prep(e, _):
            ve = vt_v[e]
            ve = jnp.where(ve < 0.1, MAX_SPIKE_TIME, ve)
            vt_v[e] = ve
            cnt_v[pl.ds(e * LANES, LANES)] = zero16f
            return 0

        lax.fori_loop(0, F, prep, 0, unroll=4)

        def rank_body(e, _):
            ke = vt_v[e]
            acc = zero16
            for f in range(F):
                acc = acc + jnp.where(vt_v[f] < ke, 1, 0).astype(jnp.int32)
            cidx = acc * LANES + lane
            c = plsc.load_gather(cnt_v, [cidx])
            plsc.store_scatter(cnt_v, [cidx], c + 1.0)
            ridx = (acc + c.astype(jnp.int32)) * LANES + lane
            plsc.store_scatter(ord_v, [ridx],
                               jnp.full((LANES,), e, jnp.int32)
                               .astype(jnp.float32))
            plsc.store_scatter(vs_v, [ridx], vt_v[e])
            return 0

        lax.fori_loop(0, F, rank_body, 0)
        vs_v[pl.ds(F * LANES, LANES)] = jnp.full((LANES,), 1.0, jnp.float32)

        for q in range(OUT_CH // QCH):
            def step(i, carry):
                ws, iws, mn = carry
                ov = ord_v[pl.ds(i * LANES, LANES)].astype(jnp.int32)
                vs = vs_v[pl.ds(i * LANES, LANES)]
                nx = vs_v[pl.ds(i * LANES + LANES, LANES)]
                base = ov * OUT_CH + (q * QCH)
                nws, niws, nmn = [], [], []
                for o in range(QCH):
                    w = plsc.load_gather(wq_v, [base + o])
                    wsum = ws[o] + w
                    isum = iws[o] + vs * w
                    d = jnp.maximum(wsum - TH, 1e-10)
                    oa = isum / d
                    cand = jnp.where(wsum < TH, MAX_SPIKE_TIME, oa)
                    cand = jnp.where(cand < vs, MAX_SPIKE_TIME, cand)
                    cand = jnp.where(cand > nx, MAX_SPIKE_TIME, cand)
                    nws.append(wsum)
                    niws.append(isum)
                    nmn.append(jnp.minimum(mn[o], cand))
                return tuple(nws), tuple(niws), tuple(nmn)

            z = tuple(jnp.zeros((LANES,), jnp.float32) for _ in range(QCH))
            m0 = tuple(jnp.full((LANES,), MAX_SPIKE_TIME, jnp.float32)
                       for _ in range(QCH))
            _, _, mn = lax.fori_loop(0, F, step, (z, z, m0))
            for o in range(QCH):
                out_v[q * QCH + o] = mn[o]

        pltpu.sync_copy(out_v, out_hbm.at[g])
        return 0

    lax.fori_loop(0, GPW, group_body, 0)


def _unfold_patches(inp):
    b, c, h, w = inp.shape
    xp = jnp.pad(inp, ((0, 0), (0, 0), (PAD, PAD), (PAD, PAD)))
    pats = []
    for i in range(K):
        for j in range(K):
            pats.append(xp[:, :, i:i + h, j:j + w])
    p = jnp.stack(pats, axis=2)
    p = p.reshape(b, c * K * K, h * w)
    return jnp.transpose(p, (0, 2, 1)).reshape(b * h * w, c * K * K)


@jax.jit
def kernel(inp, W):
    b, c, h, w = inp.shape
    l = h * w
    v = _unfold_patches(inp)
    vt = v.reshape(GROUPS, LANES, F).transpose(0, 2, 1)
    wq = _quantize(W).T.reshape(F * OUT_CH)

    mesh = plsc.VectorSubcoreMesh(core_axis_name="c", subcore_axis_name="s")
    sc = functools.partial(
        pl.kernel,
        out_type=jax.ShapeDtypeStruct((GROUPS, OUT_CH, LANES), jnp.float32),
        mesh=mesh,
        scratch_types=[
            pltpu.VMEM((F, LANES), jnp.float32),
            pltpu.VMEM((F * LANES,), jnp.float32),
            pltpu.VMEM((F * LANES,), jnp.float32),
            pltpu.VMEM(((F + 1) * LANES,), jnp.float32),
            pltpu.VMEM((F * OUT_CH,), jnp.float32),
            pltpu.VMEM((OUT_CH, LANES), jnp.float32),
        ],
        compiler_params=pltpu.CompilerParams(needs_layout_passes=False),
    )(_sc_body)
    out_t = sc(vt, wq)
    out = out_t.transpose(0, 2, 1).reshape(b, l, OUT_CH)
    return jnp.transpose(out, (0, 2, 1)).reshape(b, OUT_CH, h, w)


# speedup vs baseline: 6.2278x; 1.0020x over previous
import functools

import jax
import jax.numpy as jnp
from jax import lax
from jax.experimental import pallas as pl
from jax.experimental.pallas import tpu as pltpu
from jax.experimental.pallas import tpu_sc as plsc

MAX_SPIKE_TIME = 100.0
TH = 1.0
BIT = 8
K = 3
PAD = 1
IN_CH = 16
OUT_CH = 32
F = K * K * IN_CH
LANES = 16
NW = 32
NPOS = 2 * 32 * 32
GROUPS = NPOS // LANES
GPW = GROUPS // NW
QCH = 8


def _quantize(w):
    w = jnp.tanh(w)
    alpha = jnp.max(jnp.abs(w))
    q = 2.0 ** (BIT - 1) - 1.0
    w = jnp.clip(w / alpha, -1.0, 1.0) * q
    return jnp.round(w) * alpha / q


def _sc_body(vt_hbm, wq_hbm, out_hbm, vt_v, cnt_v, ord_v, vs_v, wq_v,
             out_v):
    cid = lax.axis_index("c")
    sid = lax.axis_index("s")
    wid = sid * 2 + cid
    pltpu.sync_copy(wq_hbm, wq_v)
    lane = lax.iota(jnp.int32, LANES)
    zero16 = jnp.zeros((LANES,), jnp.int32)
    zero16f = jnp.zeros((LANES,), jnp.float32)

    def group_body(j, _):
        g = wid * GPW + j
        pltpu.sync_copy(vt_hbm.at[g], vt_v)

        def prep(e, _):
            ve = vt_v[e]
            ve = jnp.where(ve < 0.1, MAX_SPIKE_TIME, ve)
            vt_v[e] = ve
            cnt_v[pl.ds(e * LANES, LANES)] = zero16f
            return 0

        lax.fori_loop(0, F, prep, 0, unroll=4)

        def rank_body(e, _):
            ke = vt_v[e]
            accs = [zero16, zero16, zero16, zero16]
            for f in range(F):
                accs[f % 4] = accs[f % 4] + (vt_v[f] < ke).astype(jnp.int32)
            acc = (accs[0] + accs[1]) + (accs[2] + accs[3])
            cidx = acc * LANES + lane
            c = plsc.load_gather(cnt_v, [cidx])
            plsc.store_scatter(cnt_v, [cidx], c + 1.0)
            ridx = (acc + c.astype(jnp.int32)) * LANES + lane
            plsc.store_scatter(ord_v, [ridx],
                               jnp.full((LANES,), e, jnp.int32)
                               .astype(jnp.float32))
            plsc.store_scatter(vs_v, [ridx], vt_v[e])
            return 0

        lax.fori_loop(0, F, rank_body, 0)
        vs_v[pl.ds(F * LANES, LANES)] = jnp.full((LANES,), 1.0, jnp.float32)

        for q in range(OUT_CH // QCH):
            def step(i, carry):
                ws, iws, mn = carry
                ov = ord_v[pl.ds(i * LANES, LANES)].astype(jnp.int32)
                vs = vs_v[pl.ds(i * LANES, LANES)]
                nx = vs_v[pl.ds(i * LANES + LANES, LANES)]
                base = ov * OUT_CH + (q * QCH)
                nws, niws, nmn = [], [], []
                for o in range(QCH):
                    w = plsc.load_gather(wq_v, [base + o])
                    wsum = ws[o] + w
                    isum = iws[o] + vs * w
                    d = jnp.maximum(wsum - TH, 1e-10)
                    oa = isum / d
                    cand = jnp.where(wsum < TH, MAX_SPIKE_TIME, oa)
                    cand = jnp.where(cand < vs, MAX_SPIKE_TIME, cand)
                    cand = jnp.where(cand > nx, MAX_SPIKE_TIME, cand)
                    nws.append(wsum)
                    niws.append(isum)
                    nmn.append(jnp.minimum(mn[o], cand))
                return tuple(nws), tuple(niws), tuple(nmn)

            z = tuple(jnp.zeros((LANES,), jnp.float32) for _ in range(QCH))
            m0 = tuple(jnp.full((LANES,), MAX_SPIKE_TIME, jnp.float32)
                       for _ in range(QCH))
            _, _, mn = lax.fori_loop(0, F, step, (z, z, m0))
            for o in range(QCH):
                out_v[q * QCH + o] = mn[o]

        pltpu.sync_copy(out_v, out_hbm.at[g])
        return 0

    lax.fori_loop(0, GPW, group_body, 0)


def _unfold_patches(inp):
    b, c, h, w = inp.shape
    xp = jnp.pad(inp, ((0, 0), (0, 0), (PAD, PAD), (PAD, PAD)))
    pats = []
    for i in range(K):
        for j in range(K):
            pats.append(xp[:, :, i:i + h, j:j + w])
    p = jnp.stack(pats, axis=2)
    p = p.reshape(b, c * K * K, h * w)
    return jnp.transpose(p, (0, 2, 1)).reshape(b * h * w, c * K * K)


@jax.jit
def kernel(inp, W):
    b, c, h, w = inp.shape
    l = h * w
    v = _unfold_patches(inp)
    vt = v.reshape(GROUPS, LANES, F).transpose(0, 2, 1)
    wq = _quantize(W).T.reshape(F * OUT_CH)

    mesh = plsc.VectorSubcoreMesh(core_axis_name="c", subcore_axis_name="s")
    sc = functools.partial(
        pl.kernel,
        out_type=jax.ShapeDtypeStruct((GROUPS, OUT_CH, LANES), jnp.float32),
        mesh=mesh,
        scratch_types=[
            pltpu.VMEM((F, LANES), jnp.float32),
            pltpu.VMEM((F * LANES,), jnp.float32),
            pltpu.VMEM((F * LANES,), jnp.float32),
            pltpu.VMEM(((F + 1) * LANES,), jnp.float32),
            pltpu.VMEM((F * OUT_CH,), jnp.float32),
            pltpu.VMEM((OUT_CH, LANES), jnp.float32),
        ],
        compiler_params=pltpu.CompilerParams(needs_layout_passes=False),
    )(_sc_body)
    out_t = sc(vt, wq)
    out = out_t.transpose(0, 2, 1).reshape(b, l, OUT_CH)
    return jnp.transpose(out, (0, 2, 1)).reshape(b, OUT_CH, h, w)


# speedup vs baseline: 6.3292x; 1.0163x over previous
import functools

import jax
import jax.numpy as jnp
from jax import lax
from jax.experimental import pallas as pl
from jax.experimental.pallas import tpu as pltpu
from jax.experimental.pallas import tpu_sc as plsc

MAX_SPIKE_TIME = 100.0
TH = 1.0
BIT = 8
K = 3
PAD = 1
IN_CH = 16
OUT_CH = 32
F = K * K * IN_CH
LANES = 16
NW = 32
NPOS = 2 * 32 * 32
GROUPS = NPOS // LANES
GPW = GROUPS // NW
QCH = 16


def _quantize(w):
    w = jnp.tanh(w)
    alpha = jnp.max(jnp.abs(w))
    q = 2.0 ** (BIT - 1) - 1.0
    w = jnp.clip(w / alpha, -1.0, 1.0) * q
    return jnp.round(w) * alpha / q


def _sc_body(vt_hbm, wq_hbm, out_hbm, vt_v, cnt_v, ord_v, vs_v, wq_v,
             out_v):
    cid = lax.axis_index("c")
    sid = lax.axis_index("s")
    wid = sid * 2 + cid
    pltpu.sync_copy(wq_hbm, wq_v)
    lane = lax.iota(jnp.int32, LANES)
    zero16 = jnp.zeros((LANES,), jnp.int32)
    zero16f = jnp.zeros((LANES,), jnp.float32)

    def group_body(j, _):
        g = wid * GPW + j
        pltpu.sync_copy(vt_hbm.at[g], vt_v)

        def prep(e, nlt):
            ve = vt_v[e]
            ve = jnp.where(ve < 0.1, MAX_SPIKE_TIME, ve)
            vt_v[e] = ve
            cnt_v[pl.ds(e * LANES, LANES)] = zero16f
            return nlt + (ve < MAX_SPIKE_TIME).astype(jnp.int32)

        nlt = lax.fori_loop(0, F, prep, zero16, unroll=4)
        ncut = jnp.max(nlt)

        def rank_body(e, _):
            ke = vt_v[e]
            accs = [zero16, zero16, zero16, zero16]
            for f in range(F):
                accs[f % 4] = accs[f % 4] + (vt_v[f] < ke).astype(jnp.int32)
            acc = (accs[0] + accs[1]) + (accs[2] + accs[3])
            cidx = acc * LANES + lane
            c = plsc.load_gather(cnt_v, [cidx])
            plsc.store_scatter(cnt_v, [cidx], c + 1.0)
            ridx = (acc + c.astype(jnp.int32)) * LANES + lane
            plsc.store_scatter(ord_v, [ridx],
                               jnp.full((LANES,), e, jnp.int32)
                               .astype(jnp.float32))
            plsc.store_scatter(vs_v, [ridx], vt_v[e])
            return 0

        lax.fori_loop(0, F, rank_body, 0)
        vs_v[pl.ds(F * LANES, LANES)] = jnp.full((LANES,), 1.0, jnp.float32)

        for q in range(OUT_CH // QCH):
            def step(i, carry):
                ws, iws, mn = carry
                ov = ord_v[pl.ds(i * LANES, LANES)].astype(jnp.int32)
                vs = vs_v[pl.ds(i * LANES, LANES)]
                nx = vs_v[pl.ds(i * LANES + LANES, LANES)]
                base = ov * OUT_CH + (q * QCH)
                nws, niws, nmn = [], [], []
                for o in range(QCH):
                    w = plsc.load_gather(wq_v, [base + o])
                    wsum = ws[o] + w
                    isum = iws[o] + vs * w
                    d = jnp.maximum(wsum - TH, 1e-10)
                    oa = isum / d
                    cand = jnp.where(wsum < TH, MAX_SPIKE_TIME, oa)
                    cand = jnp.where(cand < vs, MAX_SPIKE_TIME, cand)
                    cand = jnp.where(cand > nx, MAX_SPIKE_TIME, cand)
                    nws.append(wsum)
                    niws.append(isum)
                    nmn.append(jnp.minimum(mn[o], cand))
                return tuple(nws), tuple(niws), tuple(nmn)

            z = tuple(jnp.zeros((LANES,), jnp.float32) for _ in range(QCH))
            m0 = tuple(jnp.full((LANES,), MAX_SPIKE_TIME, jnp.float32)
                       for _ in range(QCH))
            _, _, mn = lax.fori_loop(0, ncut, step, (z, z, m0))
            for o in range(QCH):
                out_v[q * QCH + o] = mn[o]

        pltpu.sync_copy(out_v, out_hbm.at[g])
        return 0

    lax.fori_loop(0, GPW, group_body, 0)


def _unfold_patches(inp):
    b, c, h, w = inp.shape
    xp = jnp.pad(inp, ((0, 0), (0, 0), (PAD, PAD), (PAD, PAD)))
    pats = []
    for i in range(K):
        for j in range(K):
            pats.append(xp[:, :, i:i + h, j:j + w])
    p = jnp.stack(pats, axis=2)
    p = p.reshape(b, c * K * K, h * w)
    return jnp.transpose(p, (0, 2, 1)).reshape(b * h * w, c * K * K)


@jax.jit
def kernel(inp, W):
    b, c, h, w = inp.shape
    l = h * w
    v = _unfold_patches(inp)
    vt = v.reshape(GROUPS, LANES, F).transpose(0, 2, 1)
    wq = _quantize(W).T.reshape(F * OUT_CH)

    mesh = plsc.VectorSubcoreMesh(core_axis_name="c", subcore_axis_name="s")
    sc = functools.partial(
        pl.kernel,
        out_type=jax.ShapeDtypeStruct((GROUPS, OUT_CH, LANES), jnp.float32),
        mesh=mesh,
        scratch_types=[
            pltpu.VMEM((F, LANES), jnp.float32),
            pltpu.VMEM((F * LANES,), jnp.float32),
            pltpu.VMEM((F * LANES,), jnp.float32),
            pltpu.VMEM(((F + 1) * LANES,), jnp.float32),
            pltpu.VMEM((F * OUT_CH,), jnp.float32),
            pltpu.VMEM((OUT_CH, LANES), jnp.float32),
        ],
        compiler_params=pltpu.CompilerParams(needs_layout_passes=False),
    )(_sc_body)
    out_t = sc(vt, wq)
    out = out_t.transpose(0, 2, 1).reshape(b, l, OUT_CH)
    return jnp.transpose(out, (0, 2, 1)).reshape(b, OUT_CH, h, w)
